# baseline (speedup 1.0000x reference)
import functools

import jax
import jax.numpy as jnp
from jax import lax
from jax.experimental import pallas as pl
from jax.experimental.pallas import tpu as pltpu
from jax.experimental.pallas import tpu_sc as plsc

NC = 2
NS = 16
NW = NC * NS
CH = 128
LANES = 16
NBUF = 2


def _mm_body(x_ref, t_ref, o0_ref, o1_ref, o2_ref):
    y = lax.dot_general(x_ref[...], t_ref[...], (((0,), (0,)), ((), ())),
                        preferred_element_type=jnp.float32)
    c = o0_ref.shape[1]
    o0_ref[...] = y[:, 0:c]
    o1_ref[...] = y[:, c:2 * c]
    o2_ref[...] = y[:, 2 * c:3 * c]


def _mm3(x2d, thetaT):
    cin, m = x2d.shape
    ck3 = thetaT.shape[1]
    c = ck3 // 3
    out = jax.ShapeDtypeStruct((m, c), jnp.float32)
    return pl.pallas_call(
        _mm_body,
        grid=(1,),
        in_specs=[
            pl.BlockSpec((cin, m), lambda i: (0, 0)),
            pl.BlockSpec((cin, ck3), lambda i: (0, 0)),
        ],
        out_specs=[pl.BlockSpec((m, c), lambda i: (0, 0))] * 3,
        out_shape=[out, out, out],
    )(x2d, thetaT)


def _add2_body(a_ref, b_ref, d_ref, o_ref):
    o_ref[...] = a_ref[...] + b_ref[...] + d_ref[...]


def _add2(a, b, brow, bm=2000):
    m, ch = a.shape
    spec = pl.BlockSpec((bm, ch), lambda i: (i, 0))
    return pl.pallas_call(
        _add2_body,
        grid=(m // bm,),
        in_specs=[spec, spec, pl.BlockSpec((1, ch), lambda i: (0, 0))],
        out_specs=spec,
        out_shape=jax.ShapeDtypeStruct((m, ch), jnp.float32),
    )(a, b, brow)


def _vgather(vec, idx16):
    dnums = lax.GatherDimensionNumbers(
        offset_dims=(), collapsed_slice_dims=(0,), start_index_map=(0,))
    return lax.gather(vec, idx16[:, None], dnums, (1,),
                      mode=lax.GatherScatterMode.PROMISE_IN_BOUNDS)


def _spmm_partials(packed, table, init):
    nchunks = packed.shape[0]
    m, c = table.shape
    nt = nchunks // NW
    assert nt % NBUF == 0
    rpt = 8 * (m // 8 // NS)
    rem = m - NS * rpt

    mesh = plsc.VectorSubcoreMesh(core_axis_name="c", subcore_axis_name="s")

    scratch = (
        [pltpu.VMEM((CH, c), jnp.float32) for _ in range(NBUF)]
        + [pltpu.VMEM((3, CH), jnp.int32) for _ in range(NBUF)]
        + [pltpu.VMEM_SHARED((m, c), jnp.float32)]
        + [pltpu.SemaphoreType.DMA] * NBUF
    )

    @functools.partial(
        pl.kernel,
        out_type=jax.ShapeDtypeStruct((NC, m, c), jnp.float32),
        mesh=mesh,
        scratch_types=scratch,
    )
    def spmm(packed_hbm, table_hbm, init_hbm, out_hbm, *sc):
        gath = sc[0:NBUF]
        pkt = sc[NBUF:2 * NBUF]
        acc = sc[2 * NBUF]
        semg = sc[2 * NBUF + 1:2 * NBUF + 1 + NBUF]

        cid = lax.axis_index("c")
        sid = lax.axis_index("s")
        wid = cid * NS + sid
        cbase = wid * nt

        pltpu.sync_copy(init_hbm.at[cid, pl.ds(sid * rpt, rpt)],
                        acc.at[pl.ds(sid * rpt, rpt)])
        if rem:
            @pl.when(sid == NS - 1)
            def _():
                pltpu.sync_copy(init_hbm.at[cid, pl.ds(NS * rpt, rem)],
                                acc.at[pl.ds(NS * rpt, rem)])
        plsc.subcore_barrier()

        def fetch_pkt(t, b):
            pltpu.sync_copy(packed_hbm.at[cbase + t], pkt[b])

        def fire_gather(b):
            pltpu.async_copy(table_hbm.at[pkt[b].at[1]], gath[b], semg[b])

        def wait_gather(b):
            pltpu.make_async_copy(table_hbm.at[pkt[b].at[1]], gath[b],
                                  semg[b]).wait()

        def scatter(b):
            pltpu.sync_copy(gath[b], acc.at[pkt[b].at[0]], add=True)

        def scale(b):
            def scale_block(eb, cc):
                vbits = pkt[b][2, pl.ds(eb * LANES, LANES)]
                vblock = lax.bitcast_convert_type(vbits, jnp.float32)
                for l in range(LANES):
                    vv = _vgather(vblock, jnp.full((LANES,), l, jnp.int32))
                    e = eb * LANES + l
                    for j in range(c // LANES):
                        g = gath[b][e, pl.ds(j * LANES, LANES)]
                        gath[b][e, pl.ds(j * LANES, LANES)] = g * vv
                return cc

            lax.fori_loop(0, CH // LANES, scale_block, 0)

        fetch_pkt(0, 0)
        fire_gather(0)

        def group(outer, carry):
            for g in range(NBUF):
                t = outer * NBUF + g
                b = g
                nb = (g + 1) % NBUF

                @pl.when(t + 1 < nt)
                def _():
                    fetch_pkt(t + 1, nb)
                    fire_gather(nb)

                wait_gather(b)
                scale(b)
                scatter(b)
            return carry

        lax.fori_loop(0, nt // NBUF, group, 0)
        plsc.subcore_barrier()

        pltpu.sync_copy(acc.at[pl.ds(sid * rpt, rpt)],
                        out_hbm.at[cid, pl.ds(sid * rpt, rpt)])
        if rem:
            @pl.when(sid == NS - 1)
            def _():
                pltpu.sync_copy(acc.at[pl.ds(NS * rpt, rem)],
                                out_hbm.at[cid, pl.ds(NS * rpt, rem)])

    return spmm(packed, table, init)


def kernel(L_indices, L_values, x, theta, bias):
    rows = L_indices[0].astype(jnp.int32)
    cols = L_indices[1].astype(jnp.int32)
    vals = L_values.astype(jnp.float32)

    cout, cin, k = theta.shape
    m = x.shape[2]
    nnz = vals.shape[0]

    quant = NW * CH * NBUF
    nnzp = ((nnz + quant - 1) // quant) * quant
    pad = nnzp - nnz
    if pad:
        rows = jnp.concatenate([rows, jnp.zeros((pad,), jnp.int32)])
        cols = jnp.concatenate([cols, jnp.zeros((pad,), jnp.int32)])
        vals = jnp.concatenate([vals, jnp.zeros((pad,), jnp.float32)])
    vbits = lax.bitcast_convert_type(vals, jnp.int32)
    packed = jnp.stack([rows.reshape(-1, CH), cols.reshape(-1, CH),
                        vbits.reshape(-1, CH)], axis=1)

    thetaT = jnp.transpose(theta, (1, 2, 0)).reshape(cin, k * cout)
    z0, z1, z2 = _mm3(x[0], thetaT)

    zrow = jnp.zeros((1, cout), jnp.float32)
    biasT = bias[0, :, 0][None, :]
    zeros_mc = jnp.zeros_like(z1)

    u_p = _spmm_partials(packed, z2, jnp.stack([z1, zeros_mc]))
    u = _add2(u_p[0], u_p[1], zrow)
    y_p = _spmm_partials(packed, u, jnp.stack([z0, zeros_mc]))
    yT = _add2(y_p[0], y_p[1], biasT)
    return yT.T[None]


# speedup vs baseline: 1.1911x; 1.1911x over previous
import functools

import jax
import jax.numpy as jnp
from jax import lax
from jax.experimental import pallas as pl
from jax.experimental.pallas import tpu as pltpu
from jax.experimental.pallas import tpu_sc as plsc

NC = 2
NS = 16
NW = NC * NS
CH = 128
LANES = 16


def _mm_body(x_ref, t_ref, o0_ref, o1_ref, o2_ref):
    y = lax.dot_general(x_ref[...], t_ref[...], (((0,), (0,)), ((), ())),
                        preferred_element_type=jnp.float32)
    c = o0_ref.shape[1]
    o0_ref[...] = y[:, 0:c]
    o1_ref[...] = y[:, c:2 * c]
    o2_ref[...] = y[:, 2 * c:3 * c]


def _mm3(x2d, thetaT):
    cin, m = x2d.shape
    ck3 = thetaT.shape[1]
    c = ck3 // 3
    out = jax.ShapeDtypeStruct((m, c), jnp.float32)
    return pl.pallas_call(
        _mm_body,
        grid=(1,),
        in_specs=[
            pl.BlockSpec((cin, m), lambda i: (0, 0)),
            pl.BlockSpec((cin, ck3), lambda i: (0, 0)),
        ],
        out_specs=[pl.BlockSpec((m, c), lambda i: (0, 0))] * 3,
        out_shape=[out, out, out],
    )(x2d, thetaT)


def _add2_body(a_ref, b_ref, d_ref, o_ref):
    o_ref[...] = a_ref[...] + b_ref[...] + d_ref[...]


def _add2(a, b, brow, bm=2000):
    m, ch = a.shape
    spec = pl.BlockSpec((bm, ch), lambda i: (i, 0))
    return pl.pallas_call(
        _add2_body,
        grid=(m // bm,),
        in_specs=[spec, spec, pl.BlockSpec((1, ch), lambda i: (0, 0))],
        out_specs=spec,
        out_shape=jax.ShapeDtypeStruct((m, ch), jnp.float32),
    )(a, b, brow)


def _vgather(vec, idx16):
    dnums = lax.GatherDimensionNumbers(
        offset_dims=(), collapsed_slice_dims=(0,), start_index_map=(0,))
    return lax.gather(vec, idx16[:, None], dnums, (1,),
                      mode=lax.GatherScatterMode.PROMISE_IN_BOUNDS)


def _spmm_partials(packed, table, init):
    nchunks = packed.shape[0]
    m, c = table.shape
    nt = nchunks // NW
    rpt = 8 * (m // 8 // NS)
    rem = m - NS * rpt

    mesh = plsc.VectorSubcoreMesh(core_axis_name="c", subcore_axis_name="s")

    scratch = [
        pltpu.VMEM((CH, c), jnp.float32),
        pltpu.VMEM((3, CH), jnp.int32),
        pltpu.VMEM_SHARED((m, c), jnp.float32),
        pltpu.SemaphoreType.DMA,
    ]

    @functools.partial(
        pl.kernel,
        out_type=jax.ShapeDtypeStruct((NC, m, c), jnp.float32),
        mesh=mesh,
        scratch_types=scratch,
    )
    def spmm(packed_hbm, table_hbm, init_hbm, out_hbm, gath, pkt, acc, sem):
        cid = lax.axis_index("c")
        sid = lax.axis_index("s")
        wid = cid * NS + sid
        cbase = wid * nt

        pltpu.sync_copy(init_hbm.at[cid, pl.ds(sid * rpt, rpt)],
                        acc.at[pl.ds(sid * rpt, rpt)])
        if rem:
            @pl.when(sid == NS - 1)
            def _():
                pltpu.sync_copy(init_hbm.at[cid, pl.ds(NS * rpt, rem)],
                                acc.at[pl.ds(NS * rpt, rem)])
        plsc.subcore_barrier()

        def chunk_body(t, carry):
            pltpu.sync_copy(packed_hbm.at[cbase + t], pkt)
            pltpu.async_copy(table_hbm.at[pkt.at[1]], gath, sem).wait()

            def scale_block(eb, cc):
                vbits = pkt[2, pl.ds(eb * LANES, LANES)]
                vblock = lax.bitcast_convert_type(vbits, jnp.float32)
                for l in range(LANES):
                    vv = _vgather(vblock, jnp.full((LANES,), l, jnp.int32))
                    e = eb * LANES + l
                    for j in range(c // LANES):
                        g = gath[e, pl.ds(j * LANES, LANES)]
                        gath[e, pl.ds(j * LANES, LANES)] = g * vv
                return cc

            lax.fori_loop(0, CH // LANES, scale_block, 0)
            pltpu.sync_copy(gath, acc.at[pkt.at[0]], add=True)
            return carry

        lax.fori_loop(0, nt, chunk_body, 0)
        plsc.subcore_barrier()

        pltpu.sync_copy(acc.at[pl.ds(sid * rpt, rpt)],
                        out_hbm.at[cid, pl.ds(sid * rpt, rpt)])
        if rem:
            @pl.when(sid == NS - 1)
            def _():
                pltpu.sync_copy(acc.at[pl.ds(NS * rpt, rem)],
                                out_hbm.at[cid, pl.ds(NS * rpt, rem)])

    return spmm(packed, table, init)


def kernel(L_indices, L_values, x, theta, bias):
    rows = L_indices[0].astype(jnp.int32)
    cols = L_indices[1].astype(jnp.int32)
    vals = L_values.astype(jnp.float32)

    cout, cin, k = theta.shape
    nnz = vals.shape[0]

    quant = NW * CH
    nnzp = ((nnz + quant - 1) // quant) * quant
    pad = nnzp - nnz
    if pad:
        rows = jnp.concatenate([rows, jnp.zeros((pad,), jnp.int32)])
        cols = jnp.concatenate([cols, jnp.zeros((pad,), jnp.int32)])
        vals = jnp.concatenate([vals, jnp.zeros((pad,), jnp.float32)])
    vbits = lax.bitcast_convert_type(vals, jnp.int32)
    packed = jnp.stack([rows.reshape(-1, CH), cols.reshape(-1, CH),
                        vbits.reshape(-1, CH)], axis=1)

    thetaT = jnp.transpose(theta, (1, 2, 0)).reshape(cin, k * cout)
    biasT = bias[0, :, 0][None, :]
    zrow = jnp.zeros((1, cout), jnp.float32)
    zeros_mc = jnp.zeros((x.shape[2], cout), jnp.float32)

    z0, z1, z2 = _mm3(x[0], thetaT)

    u_p = _spmm_partials(packed, z2, jnp.stack([z1, zeros_mc]))
    u = _add2(u_p[0], u_p[1], zrow)
    y_p = _spmm_partials(packed, u, jnp.stack([z0, zeros_mc]))
    yT = _add2(y_p[0], y_p[1], biasT)
    return yT.T[None]
